# baseline (speedup 1.0000x reference)
import functools
import numpy as np
import jax
import jax.numpy as jnp
from jax.experimental import pallas as pl
from jax.experimental.pallas import tpu as pltpu

N_ATOMS = 10000
N_CG = 1000
N_EDGES = 320000
N_CG_EDGES = 32000
N_ATOM_BASIS = 128
N_FILTERS = 128
N_GAUSSIANS = 50
N_CONV = 3
CUTOFF = 5.0

_OFFSETS = np.linspace(0.0, CUTOFF, N_GAUSSIANS).astype(np.float32)
_WIDTH = float(_OFFSETS[1] - _OFFSETS[0])
_COEFF = -0.5 / _WIDTH**2
_LOG2 = float(np.log(2.0))


def _ssp(x):
    return jnp.logaddexp(x, 0.0) - _LOG2


def _embed_body(z_ref, emb_ref, out_ref):
    z = z_ref[...]
    oh = (z == jax.lax.broadcasted_iota(jnp.int32, (1, 100), 1)).astype(jnp.float32)
    out_ref[...] = jnp.dot(oh, emb_ref[...], preferred_element_type=jnp.float32)


def _embed(z2d, embed):
    n = z2d.shape[0]
    blk = 1000
    return pl.pallas_call(
        _embed_body,
        grid=(n // blk,),
        in_specs=[
            pl.BlockSpec((blk, 1), lambda i: (i, 0)),
            pl.BlockSpec((100, N_ATOM_BASIS), lambda i: (0, 0)),
        ],
        out_specs=pl.BlockSpec((blk, N_ATOM_BASIS), lambda i: (i, 0)),
        out_shape=jax.ShapeDtypeStruct((n, N_ATOM_BASIS), jnp.float32),
    )(z2d, embed)


def _counts_body(m_ref, out_ref):
    i = pl.program_id(0)
    m = m_ref[...]
    oh = (m == jax.lax.broadcasted_iota(jnp.int32, (1, N_CG), 1)).astype(jnp.float32)
    c = jnp.sum(oh, axis=0, keepdims=True)

    @pl.when(i == 0)
    def _():
        out_ref[...] = jnp.zeros_like(out_ref)

    out_ref[...] += c


def _counts(m2d):
    n = m2d.shape[0]
    blk = 1000
    return pl.pallas_call(
        _counts_body,
        grid=(n // blk,),
        in_specs=[pl.BlockSpec((blk, 1), lambda i: (i, 0))],
        out_specs=pl.BlockSpec((1, N_CG), lambda i: (0, 0)),
        out_shape=jax.ShapeDtypeStruct((1, N_CG), jnp.float32),
    )(m2d)


def _edge_filter_body(d2_ref, W1_ref, b1_ref, W2_ref, b2_ref, out_ref):
    d = jnp.sqrt(d2_ref[...])
    offs = jax.lax.broadcasted_iota(jnp.int32, (1, N_GAUSSIANS), 1).astype(jnp.float32) * (CUTOFF / (N_GAUSSIANS - 1))
    g = jnp.exp(_COEFF * (d - offs) ** 2)
    h = _ssp(jnp.dot(g, W1_ref[...], preferred_element_type=jnp.float32) + b1_ref[...])
    out_ref[...] = jnp.dot(h, W2_ref[...], preferred_element_type=jnp.float32) + b2_ref[...]


def _edge_filter(d2, W1, b1, W2, b2):
    e = d2.shape[0]
    blk = 2000
    return pl.pallas_call(
        _edge_filter_body,
        grid=(e // blk,),
        in_specs=[
            pl.BlockSpec((blk, 1), lambda i: (i, 0)),
            pl.BlockSpec((N_GAUSSIANS, N_GAUSSIANS), lambda i: (0, 0)),
            pl.BlockSpec((1, N_GAUSSIANS), lambda i: (0, 0)),
            pl.BlockSpec((N_GAUSSIANS, N_FILTERS), lambda i: (0, 0)),
            pl.BlockSpec((1, N_FILTERS), lambda i: (0, 0)),
        ],
        out_specs=pl.BlockSpec((blk, N_FILTERS), lambda i: (i, 0)),
        out_shape=jax.ShapeDtypeStruct((e, N_FILTERS), jnp.float32),
    )(d2, W1, b1, W2, b2)


def _rn_body(s_ref, W_ref, b_ref, out_ref):
    out_ref[...] = jnp.dot(s_ref[...], W_ref[...], preferred_element_type=jnp.float32) + b_ref[...]


def _rn(s, W, b):
    n = s.shape[0]
    blk = min(n, 2000)
    return pl.pallas_call(
        _rn_body,
        grid=(n // blk,),
        in_specs=[
            pl.BlockSpec((blk, N_ATOM_BASIS), lambda i: (i, 0)),
            pl.BlockSpec((N_ATOM_BASIS, N_FILTERS), lambda i: (0, 0)),
            pl.BlockSpec((1, N_FILTERS), lambda i: (0, 0)),
        ],
        out_specs=pl.BlockSpec((blk, N_FILTERS), lambda i: (i, 0)),
        out_shape=jax.ShapeDtypeStruct((n, N_FILTERS), jnp.float32),
    )(s, W, b)


def _update_body(s_ref, a0_ref, a1_ref, W1_ref, b1_ref, W2_ref, b2_ref, out_ref):
    agg = a0_ref[...] + a1_ref[...]
    h = _ssp(jnp.dot(agg, W1_ref[...], preferred_element_type=jnp.float32) + b1_ref[...])
    out_ref[...] = s_ref[...] + jnp.dot(h, W2_ref[...], preferred_element_type=jnp.float32) + b2_ref[...]


def _update(s, a0, a1, Wu1, bu1, Wu2, bu2):
    n = s.shape[0]
    blk = min(n, 2000)
    return pl.pallas_call(
        _update_body,
        grid=(n // blk,),
        in_specs=[
            pl.BlockSpec((blk, N_ATOM_BASIS), lambda i: (i, 0)),
            pl.BlockSpec((blk, N_FILTERS), lambda i: (i, 0)),
            pl.BlockSpec((blk, N_FILTERS), lambda i: (i, 0)),
            pl.BlockSpec((N_FILTERS, N_ATOM_BASIS), lambda i: (0, 0)),
            pl.BlockSpec((1, N_ATOM_BASIS), lambda i: (0, 0)),
            pl.BlockSpec((N_ATOM_BASIS, N_ATOM_BASIS), lambda i: (0, 0)),
            pl.BlockSpec((1, N_ATOM_BASIS), lambda i: (0, 0)),
        ],
        out_specs=pl.BlockSpec((blk, N_ATOM_BASIS), lambda i: (i, 0)),
        out_shape=jax.ShapeDtypeStruct((n, N_ATOM_BASIS), jnp.float32),
    )(s, a0, a1, Wu1, bu1, Wu2, bu2)


def _div_body(S_ref, c_ref, out_ref):
    c = jnp.maximum(c_ref[...], 1.0)
    out_ref[...] = S_ref[...] / c


def _div(S, c_col):
    return pl.pallas_call(
        _div_body,
        grid=(1,),
        in_specs=[
            pl.BlockSpec((N_CG, N_ATOM_BASIS), lambda i: (0, 0)),
            pl.BlockSpec((N_CG, 1), lambda i: (0, 0)),
        ],
        out_specs=pl.BlockSpec((N_CG, N_ATOM_BASIS), lambda i: (0, 0)),
        out_shape=jax.ShapeDtypeStruct((N_CG, N_ATOM_BASIS), jnp.float32),
    )(S, c_col)


def _dist2(xyz, nbr):
    d = xyz[nbr[:, 0]] - xyz[nbr[:, 1]]
    return (d * d).sum(1)[:, None]


def _message_agg(rn, f, nbr, n):
    rij = rn[nbr[:, 0]] * f
    rji = rn[nbr[:, 1]] * f
    agg = jax.ops.segment_sum(rij, nbr[:, 1], num_segments=n) \
        + jax.ops.segment_sum(rji, nbr[:, 0], num_segments=n)
    return agg


def _seg_sum_rows(x, idx, num):
    return jax.ops.segment_sum(x, idx, num_segments=num)


def kernel(z, xyz, cg_xyz, mapping, nbr_list, CG_nbr_list, embed,
           W_ef1, b_ef1, W_ef2, b_ef2, W_nf, b_nf, W_u1, b_u1, W_u2, b_u2):
    z2d = z.astype(jnp.int32)[:, None]
    m2d = mapping.astype(jnp.int32)[:, None]

    d2_ij = _dist2(xyz, nbr_list)
    d2_IJ = _dist2(cg_xyz, CG_nbr_list)

    s_i = _embed(z2d, embed)
    counts = _counts(m2d)
    c_col = counts.T
    zero = jnp.zeros_like(s_i)

    S_I = None
    for i in range(N_CONV):
        f = _edge_filter(d2_ij, W_ef1[i], b_ef1[i][None, :], W_ef2[i], b_ef2[i][None, :])
        rn = _rn(s_i, W_nf[i], b_nf[i][None, :])
        agg = _message_agg(rn, f, nbr_list, N_ATOMS)
        s_i = _update(s_i, agg, zero, W_u1[i], b_u1[i][None, :], W_u2[i], b_u2[i][None, :])

        S_sum = _seg_sum_rows(s_i, mapping, N_CG)
        S_input = _div(S_sum, c_col)
        if i == 0:
            S_I = S_input

        j = N_CONV + i
        fc = _edge_filter(d2_IJ, W_ef1[j], b_ef1[j][None, :], W_ef2[j], b_ef2[j][None, :])
        Rn = _rn(S_input, W_nf[j], b_nf[j][None, :])
        Agg = _message_agg(Rn, fc, CG_nbr_list, N_CG)
        S_I = _update(S_I, Agg, jnp.zeros_like(S_I), W_u1[j], b_u1[j][None, :], W_u2[j], b_u2[j][None, :])

        s_i = s_i + S_I[mapping]

    return S_I


# speedup vs baseline: 2.6588x; 2.6588x over previous
import functools
import numpy as np
import jax
import jax.numpy as jnp
from jax import lax
from jax.experimental import pallas as pl
from jax.experimental.pallas import tpu as pltpu
from jax.experimental.pallas import tpu_sc as plsc

N_ATOMS = 10000
N_CG = 1000
N_EDGES = 320000
N_CG_EDGES = 32000
N_ATOM_BASIS = 128
N_FILTERS = 128
N_GAUSSIANS = 50
N_CONV = 3
CUTOFF = 5.0

_OFFSETS = np.linspace(0.0, CUTOFF, N_GAUSSIANS).astype(np.float32)
_WIDTH = float(_OFFSETS[1] - _OFFSETS[0])
_COEFF = -0.5 / _WIDTH**2
_LOG2 = float(np.log(2.0))


def _ssp(x):
    return jnp.logaddexp(x, 0.0) - _LOG2


def _embed_body(z_ref, emb_ref, out_ref):
    z = z_ref[...]
    oh = (z == jax.lax.broadcasted_iota(jnp.int32, (1, 100), 1)).astype(jnp.float32)
    out_ref[...] = jnp.dot(oh, emb_ref[...], preferred_element_type=jnp.float32)


def _embed(z2d, embed):
    n = z2d.shape[0]
    blk = 1000
    return pl.pallas_call(
        _embed_body,
        grid=(n // blk,),
        in_specs=[
            pl.BlockSpec((blk, 1), lambda i: (i, 0)),
            pl.BlockSpec((100, N_ATOM_BASIS), lambda i: (0, 0)),
        ],
        out_specs=pl.BlockSpec((blk, N_ATOM_BASIS), lambda i: (i, 0)),
        out_shape=jax.ShapeDtypeStruct((n, N_ATOM_BASIS), jnp.float32),
    )(z2d, embed)


def _counts_body(m_ref, out_ref):
    i = pl.program_id(0)
    m = m_ref[...]
    oh = (m == jax.lax.broadcasted_iota(jnp.int32, (1, N_CG), 1)).astype(jnp.float32)
    c = jnp.sum(oh, axis=0, keepdims=True)

    @pl.when(i == 0)
    def _():
        out_ref[...] = jnp.zeros_like(out_ref)

    out_ref[...] += c


def _counts(m2d):
    n = m2d.shape[0]
    blk = 1000
    return pl.pallas_call(
        _counts_body,
        grid=(n // blk,),
        in_specs=[pl.BlockSpec((blk, 1), lambda i: (i, 0))],
        out_specs=pl.BlockSpec((1, N_CG), lambda i: (0, 0)),
        out_shape=jax.ShapeDtypeStruct((1, N_CG), jnp.float32),
    )(m2d)


def _edge_filter_body(d2_ref, W1_ref, b1_ref, W2_ref, b2_ref, out_ref):
    d = jnp.sqrt(d2_ref[...])
    offs = jax.lax.broadcasted_iota(jnp.int32, (1, N_GAUSSIANS), 1).astype(jnp.float32) * (CUTOFF / (N_GAUSSIANS - 1))
    g = jnp.exp(_COEFF * (d - offs) ** 2)
    h = _ssp(jnp.dot(g, W1_ref[...], preferred_element_type=jnp.float32) + b1_ref[...])
    out_ref[...] = jnp.dot(h, W2_ref[...], preferred_element_type=jnp.float32) + b2_ref[...]


def _edge_filter(d2, W1, b1, W2, b2):
    e = d2.shape[0]
    blk = 2000
    return pl.pallas_call(
        _edge_filter_body,
        grid=(e // blk,),
        in_specs=[
            pl.BlockSpec((blk, 1), lambda i: (i, 0)),
            pl.BlockSpec((N_GAUSSIANS, N_GAUSSIANS), lambda i: (0, 0)),
            pl.BlockSpec((1, N_GAUSSIANS), lambda i: (0, 0)),
            pl.BlockSpec((N_GAUSSIANS, N_FILTERS), lambda i: (0, 0)),
            pl.BlockSpec((1, N_FILTERS), lambda i: (0, 0)),
        ],
        out_specs=pl.BlockSpec((blk, N_FILTERS), lambda i: (i, 0)),
        out_shape=jax.ShapeDtypeStruct((e, N_FILTERS), jnp.float32),
    )(d2, W1, b1, W2, b2)


def _rn_body(s_ref, W_ref, b_ref, out_ref):
    out_ref[...] = jnp.dot(s_ref[...], W_ref[...], preferred_element_type=jnp.float32) + b_ref[...]


def _rn(s, W, b):
    n = s.shape[0]
    blk = min(n, 2000)
    return pl.pallas_call(
        _rn_body,
        grid=(n // blk,),
        in_specs=[
            pl.BlockSpec((blk, N_ATOM_BASIS), lambda i: (i, 0)),
            pl.BlockSpec((N_ATOM_BASIS, N_FILTERS), lambda i: (0, 0)),
            pl.BlockSpec((1, N_FILTERS), lambda i: (0, 0)),
        ],
        out_specs=pl.BlockSpec((blk, N_FILTERS), lambda i: (i, 0)),
        out_shape=jax.ShapeDtypeStruct((n, N_FILTERS), jnp.float32),
    )(s, W, b)


def _update_body(s_ref, a0_ref, a1_ref, W1_ref, b1_ref, W2_ref, b2_ref, out_ref):
    agg = a0_ref[...] + a1_ref[...]
    h = _ssp(jnp.dot(agg, W1_ref[...], preferred_element_type=jnp.float32) + b1_ref[...])
    out_ref[...] = s_ref[...] + jnp.dot(h, W2_ref[...], preferred_element_type=jnp.float32) + b2_ref[...]


def _update(s, a0, a1, Wu1, bu1, Wu2, bu2):
    n = s.shape[0]
    blk = min(n, 2000)
    return pl.pallas_call(
        _update_body,
        grid=(n // blk,),
        in_specs=[
            pl.BlockSpec((blk, N_ATOM_BASIS), lambda i: (i, 0)),
            pl.BlockSpec((blk, N_FILTERS), lambda i: (i, 0)),
            pl.BlockSpec((blk, N_FILTERS), lambda i: (i, 0)),
            pl.BlockSpec((N_FILTERS, N_ATOM_BASIS), lambda i: (0, 0)),
            pl.BlockSpec((1, N_ATOM_BASIS), lambda i: (0, 0)),
            pl.BlockSpec((N_ATOM_BASIS, N_ATOM_BASIS), lambda i: (0, 0)),
            pl.BlockSpec((1, N_ATOM_BASIS), lambda i: (0, 0)),
        ],
        out_specs=pl.BlockSpec((blk, N_ATOM_BASIS), lambda i: (i, 0)),
        out_shape=jax.ShapeDtypeStruct((n, N_ATOM_BASIS), jnp.float32),
    )(s, a0, a1, Wu1, bu1, Wu2, bu2)


def _div_body(S_ref, c_ref, out_ref):
    c = jnp.maximum(c_ref[...], 1.0)
    out_ref[...] = S_ref[...] / c


def _div(S, c_col):
    return pl.pallas_call(
        _div_body,
        grid=(1,),
        in_specs=[
            pl.BlockSpec((N_CG, N_ATOM_BASIS), lambda i: (0, 0)),
            pl.BlockSpec((N_CG, 1), lambda i: (0, 0)),
        ],
        out_specs=pl.BlockSpec((N_CG, N_ATOM_BASIS), lambda i: (0, 0)),
        out_shape=jax.ShapeDtypeStruct((N_CG, N_ATOM_BASIS), jnp.float32),
    )(S, c_col)


_NC, _NS = 2, 16
_NW = _NC * _NS


def _sc_mesh():
    return plsc.VectorSubcoreMesh(
        core_axis_name="c", subcore_axis_name="s", num_cores=_NC, num_subcores=_NS)


def _msg_scatter_sc(rn, f, src_ch, dst_ch, zeros, n_nodes, n_chunks):
    N = n_nodes
    BR = 40
    nbl = N // BR
    kz = -(-nbl // _NS)
    kmax = -(-n_chunks // _NW)

    def body(rn_hbm, f_hbm, src_hbm, dst_hbm, z_hbm, out_hbm,
             agg_sh, idx_s, idx_d, f_v, rs_v, rd_v, sem):
        c = lax.axis_index("c")
        s = lax.axis_index("s")
        w = s * _NC + c

        def zero_blk(k2, carry):
            bid = k2 * _NS + s

            @pl.when(bid < nbl)
            def _():
                off = pl.multiple_of(bid * BR, BR)
                pltpu.sync_copy(z_hbm, agg_sh.at[pl.ds(off, BR)])

            return carry

        lax.fori_loop(0, kz, zero_blk, 0)
        plsc.subcore_barrier()

        def chunk(k, carry):
            cid = k * _NW + w

            @pl.when(cid < n_chunks)
            def _():
                pltpu.sync_copy(src_hbm.at[cid], idx_s)
                pltpu.sync_copy(dst_hbm.at[cid], idx_d)
                foff = pl.multiple_of(cid * 128, 128)
                pltpu.sync_copy(f_hbm.at[pl.ds(foff, 128)], f_v)
                d1 = pltpu.async_copy(rn_hbm.at[idx_s], rs_v, sem)
                d2 = pltpu.async_copy(rn_hbm.at[idx_d], rd_v, sem)
                d1.wait()
                d2.wait()

                def row(r, carry2):
                    for j in range(8):
                        sl = pl.ds(j * 16, 16)
                        fv = f_v[r, sl]
                        rs_v[r, sl] = rs_v[r, sl] * fv
                        rd_v[r, sl] = rd_v[r, sl] * fv
                    return carry2

                lax.fori_loop(0, 128, row, 0)
                pltpu.sync_copy(rs_v, agg_sh.at[idx_d], add=True)
                pltpu.sync_copy(rd_v, agg_sh.at[idx_s], add=True)

            return carry

        lax.fori_loop(0, kmax, chunk, 0)
        plsc.subcore_barrier()

        def read_blk(k2, carry):
            bid = k2 * _NS + s

            @pl.when(bid < nbl)
            def _():
                off = pl.multiple_of(bid * BR, BR)
                pltpu.sync_copy(agg_sh.at[pl.ds(off, BR)],
                                out_hbm.at[c, pl.ds(off, BR)])

            return carry

        lax.fori_loop(0, kz, read_blk, 0)

    return pl.kernel(
        body,
        out_type=jax.ShapeDtypeStruct((_NC, N, N_FILTERS), jnp.float32),
        mesh=_sc_mesh(),
        scratch_types=[
            pltpu.VMEM_SHARED((N, N_FILTERS), jnp.float32),
            pltpu.VMEM((128,), jnp.int32),
            pltpu.VMEM((128,), jnp.int32),
            pltpu.VMEM((128, N_FILTERS), jnp.float32),
            pltpu.VMEM((128, N_FILTERS), jnp.float32),
            pltpu.VMEM((128, N_FILTERS), jnp.float32),
            pltpu.SemaphoreType.DMA,
        ],
    )(rn, f, src_ch, dst_ch, zeros)


def _dist2(xyz, nbr):
    d = xyz[nbr[:, 0]] - xyz[nbr[:, 1]]
    return (d * d).sum(1)[:, None]


def _message_agg(rn, f, nbr, n):
    rij = rn[nbr[:, 0]] * f
    rji = rn[nbr[:, 1]] * f
    agg = jax.ops.segment_sum(rij, nbr[:, 1], num_segments=n) \
        + jax.ops.segment_sum(rji, nbr[:, 0], num_segments=n)
    return agg


def _seg_sum_rows(x, idx, num):
    return jax.ops.segment_sum(x, idx, num_segments=num)


def kernel(z, xyz, cg_xyz, mapping, nbr_list, CG_nbr_list, embed,
           W_ef1, b_ef1, W_ef2, b_ef2, W_nf, b_nf, W_u1, b_u1, W_u2, b_u2):
    z2d = z.astype(jnp.int32)[:, None]
    m2d = mapping.astype(jnp.int32)[:, None]

    d2_ij = _dist2(xyz, nbr_list)
    d2_IJ = _dist2(cg_xyz, CG_nbr_list)

    src_a = nbr_list[:, 0].astype(jnp.int32).reshape(N_EDGES // 128, 128)
    dst_a = nbr_list[:, 1].astype(jnp.int32).reshape(N_EDGES // 128, 128)
    src_c = CG_nbr_list[:, 0].astype(jnp.int32).reshape(N_CG_EDGES // 128, 128)
    dst_c = CG_nbr_list[:, 1].astype(jnp.int32).reshape(N_CG_EDGES // 128, 128)
    zeros_br = jnp.zeros((40, N_FILTERS), jnp.float32)

    s_i = _embed(z2d, embed)
    counts = _counts(m2d)
    c_col = counts.T

    S_I = None
    for i in range(N_CONV):
        f = _edge_filter(d2_ij, W_ef1[i], b_ef1[i][None, :], W_ef2[i], b_ef2[i][None, :])
        rn = _rn(s_i, W_nf[i], b_nf[i][None, :])
        agg2 = _msg_scatter_sc(rn, f, src_a, dst_a, zeros_br, N_ATOMS, N_EDGES // 128)
        s_i = _update(s_i, agg2[0], agg2[1], W_u1[i], b_u1[i][None, :], W_u2[i], b_u2[i][None, :])

        S_sum = _seg_sum_rows(s_i, mapping, N_CG)
        S_input = _div(S_sum, c_col)
        if i == 0:
            S_I = S_input

        j = N_CONV + i
        fc = _edge_filter(d2_IJ, W_ef1[j], b_ef1[j][None, :], W_ef2[j], b_ef2[j][None, :])
        Rn = _rn(S_input, W_nf[j], b_nf[j][None, :])
        Agg2 = _msg_scatter_sc(Rn, fc, src_c, dst_c, zeros_br, N_CG, N_CG_EDGES // 128)
        S_I = _update(S_I, Agg2[0], Agg2[1], W_u1[j], b_u1[j][None, :], W_u2[j], b_u2[j][None, :])

        s_i = s_i + S_I[mapping]

    return S_I
